# baseline (speedup 1.0000x reference)
import functools

import jax
import jax.numpy as jnp
from jax import lax
from jax.experimental import pallas as pl
from jax.experimental.pallas import tpu as pltpu
from jax.experimental.pallas import tpu_sc as plsc

_ALPHA = 0.9
_NUM_PARTS = 7
_FEATURE_DIM = 256


def _sc_reduce_body(embs_hbm, vm_hbm, out_hbm, ebuf, vbuf, wrow, acbuf, frow,
                    esem, vsem, *, R, T, D, V, L, NC, RPW, alpha):
    f32 = jnp.float32
    i32 = jnp.int32
    wid = lax.axis_index("s") * NC + lax.axis_index("c")
    base = wid * RPW

    VL = (V + 1) * T * L

    def start_row(j, slot):
        pltpu.make_async_copy(embs_hbm.at[base + j], ebuf.at[slot],
                              esem).start()
        pltpu.make_async_copy(vm_hbm.at[base + j],
                              vbuf.at[pl.ds(slot * VL, VL)], vsem).start()

    def process(j, slot):
        pltpu.make_async_copy(embs_hbm.at[base], ebuf.at[slot], esem).wait()
        pltpu.make_async_copy(vm_hbm.at[base],
                              vbuf.at[pl.ds(slot * VL, VL)], vsem).wait()
        vbase = slot * VL
        for p in range(V):
            def chain_step(k, v):
                t = T - 1 - k
                vis_t = vbuf[pl.ds(vbase + (p * T + t) * L, L)]
                m = vbuf[pl.ds(vbase + (V * T + t) * L, L)]
                v_nz = jnp.where(v != 0.0, 1.0, 0.0)
                d_nz = jnp.where(vis_t != 0.0, 1.0, 0.0)
                xor = v_nz + d_nz - 2.0 * v_nz * d_nz
                a_t = v * vis_t * alpha + xor * v
                c_t = v * vis_t * (1.0 - alpha) + xor * vis_t
                acbuf[pl.ds(t * L, L)] = m * a_t + (1.0 - m)
                acbuf[pl.ds((T + t) * L, L)] = m * c_t
                return m * jnp.maximum(v, vis_t) + (1.0 - m) * v
            lax.fori_loop(0, T, chain_step, jnp.zeros((L,), f32))

            def prefix_step(t, prod):
                wrow[pl.ds((p * T + t) * L, L)] = acbuf[
                    pl.ds((T + t) * L, L)] * prod
                return prod * acbuf[pl.ds(t * L, L)]
            lax.fori_loop(0, T, prefix_step, jnp.ones((L,), f32))

        FD = D // V
        CH = FD // L
        for p in range(V):
            def t_step(t, accs):
                ws = wrow[pl.ds((p * T + t) * L, L)]
                return tuple(
                    acc + ws * ebuf[slot, t, pl.ds(p * FD + c * L, L)]
                    for c, acc in enumerate(accs))
            accs = lax.fori_loop(
                0, T, t_step, tuple(jnp.zeros((L,), f32) for _ in range(CH)))
            for c in range(CH):
                frow[pl.ds(p * FD + c * L, L)] = accs[c]
        pltpu.sync_copy(frow, out_hbm.at[base + j])

    start_row(0, 0)
    start_row(1, 1)

    def row_step(g, carry):
        slot = lax.rem(g, 2)
        process(g, slot)
        @pl.when(g + 2 < RPW)
        def _():
            start_row(g + 2, slot)
        return carry
    lax.fori_loop(0, RPW, row_step, jnp.int32(0))


def _tc_linear_body(mask_ref, f_ref, w_ref, b_ref, out_ref):
    f32 = jnp.float32
    lin = jax.lax.dot_general(
        f_ref[:, :], w_ref[:, :], (((1,), (1,)), ((), ())),
        preferred_element_type=f32)
    lin = lin + b_ref[:, :]
    new_mask = jnp.max(mask_ref[:, :], axis=1, keepdims=True)
    out_ref[:, :] = jnp.where(new_mask > 0.0, lin, 0.0)


def kernel(embs, vis, masks, W, b):
    B, N, T, D = embs.shape
    V = vis.shape[-1]
    K = W.shape[0]
    R = B * N
    L = 16

    info = plsc.get_sparse_core_info()
    NC, NS = info.num_cores, info.num_subcores
    NW = NC * NS
    RPW = R // NW

    embs3 = embs.reshape(R, T, D)
    masks2 = masks.reshape(R, T).astype(jnp.float32)
    vis_b = jnp.broadcast_to(
        jnp.transpose(vis.reshape(R, T, V), (0, 2, 1))[..., None],
        (R, V, T, L))
    mask_b = jnp.broadcast_to(masks2[:, None, :, None], (R, 1, T, L))
    vm = jnp.concatenate([vis_b, mask_b], axis=1).reshape(R, (V + 1) * T * L)

    mesh = plsc.VectorSubcoreMesh(core_axis_name="c", subcore_axis_name="s")
    sc_body = functools.partial(
        _sc_reduce_body, R=R, T=T, D=D, V=V, L=L, NC=NC, RPW=RPW,
        alpha=_ALPHA)
    feats = pl.kernel(
        sc_body,
        out_type=jax.ShapeDtypeStruct((R, D), jnp.float32),
        mesh=mesh,
        scratch_types=[
            pltpu.VMEM((2, T, D), jnp.float32),
            pltpu.VMEM((2 * (V + 1) * T * L,), jnp.float32),
            pltpu.VMEM((V * T * L,), jnp.float32),
            pltpu.VMEM((2 * T * L,), jnp.float32),
            pltpu.VMEM((D,), jnp.float32),
            pltpu.SemaphoreType.DMA,
            pltpu.SemaphoreType.DMA,
        ],
    )(embs3, vm)

    GL = 256
    out = pl.pallas_call(
        _tc_linear_body,
        grid=(R // GL,),
        in_specs=[
            pl.BlockSpec((GL, T), lambda i: (i, 0)),
            pl.BlockSpec((GL, D), lambda i: (i, 0)),
            pl.BlockSpec((K, D), lambda i: (0, 0)),
            pl.BlockSpec((1, K), lambda i: (0, 0)),
        ],
        out_specs=pl.BlockSpec((GL, K), lambda i: (i, 0)),
        out_shape=jax.ShapeDtypeStruct((R, K), jnp.float32),
    )(masks2, feats, W, b.reshape(1, K))
    return out.reshape(B, N, K)


# speedup vs baseline: 1.0006x; 1.0006x over previous
import functools

import jax
import jax.numpy as jnp
from jax import lax
from jax.experimental import pallas as pl
from jax.experimental.pallas import tpu as pltpu
from jax.experimental.pallas import tpu_sc as plsc

_ALPHA = 0.9
_NUM_PARTS = 7
_FEATURE_DIM = 256


def _sc_reduce_body(embs_hbm, vm_hbm, out_hbm, ebuf, vbuf, wrow, acbuf, frow,
                    esem, vsem, *, R, T, D, V, L, NC, RPW, alpha):
    f32 = jnp.float32
    i32 = jnp.int32
    wid = lax.axis_index("s") * NC + lax.axis_index("c")
    base = wid * RPW

    VL = (V + 1) * T * L

    def start_row(j, slot):
        pltpu.make_async_copy(embs_hbm.at[base + j], ebuf.at[slot],
                              esem).start()
        pltpu.make_async_copy(vm_hbm.at[base + j],
                              vbuf.at[pl.ds(slot * VL, VL)], vsem).start()

    def process(j, slot):
        pltpu.make_async_copy(embs_hbm.at[base], ebuf.at[slot], esem).wait()
        pltpu.make_async_copy(vm_hbm.at[base],
                              vbuf.at[pl.ds(slot * VL, VL)], vsem).wait()
        vbase = slot * VL
        for p in range(V):
            def chain_step(k, v):
                t = T - 1 - k
                vis_t = vbuf[pl.ds(vbase + (p * T + t) * L, L)]
                m = vbuf[pl.ds(vbase + (V * T + t) * L, L)]
                v_nz = jnp.where(v != 0.0, 1.0, 0.0)
                d_nz = jnp.where(vis_t != 0.0, 1.0, 0.0)
                xor = v_nz + d_nz - 2.0 * v_nz * d_nz
                a_t = v * vis_t * alpha + xor * v
                c_t = v * vis_t * (1.0 - alpha) + xor * vis_t
                acbuf[pl.ds(t * L, L)] = m * a_t + (1.0 - m)
                acbuf[pl.ds((T + t) * L, L)] = m * c_t
                return m * jnp.maximum(v, vis_t) + (1.0 - m) * v
            lax.fori_loop(0, T, chain_step, jnp.zeros((L,), f32))

            def prefix_step(t, prod):
                wrow[pl.ds((p * T + t) * L, L)] = acbuf[
                    pl.ds((T + t) * L, L)] * prod
                return prod * acbuf[pl.ds(t * L, L)]
            lax.fori_loop(0, T, prefix_step, jnp.ones((L,), f32))

        FD = D // V
        CH = FD // L
        for p in range(V):
            def t_step(t, accs):
                ws = wrow[pl.ds((p * T + t) * L, L)]
                return tuple(
                    acc + ws * ebuf[slot, t, pl.ds(p * FD + c * L, L)]
                    for c, acc in enumerate(accs))
            accs = lax.fori_loop(
                0, T, t_step, tuple(jnp.zeros((L,), f32) for _ in range(CH)))
            for c in range(CH):
                frow[pl.ds(p * FD + c * L, L)] = accs[c]
        pltpu.sync_copy(frow, out_hbm.at[base + j])

    start_row(0, 0)
    start_row(1, 1)

    def row_step(g, carry):
        slot = lax.rem(g, 2)
        process(g, slot)
        @pl.when(g + 2 < RPW)
        def _():
            start_row(g + 2, slot)
        return carry
    lax.fori_loop(0, RPW, row_step, jnp.int32(0))


def _tc_linear_body(mask_ref, f_ref, w_ref, b_ref, out_ref):
    f32 = jnp.float32
    lin = jax.lax.dot_general(
        f_ref[:, :], w_ref[:, :], (((1,), (1,)), ((), ())),
        preferred_element_type=f32)
    lin = lin + b_ref[:, :]
    new_mask = jnp.max(mask_ref[:, :], axis=1, keepdims=True)
    out_ref[:, :] = jnp.where(new_mask > 0.0, lin, 0.0)


def kernel(embs, vis, masks, W, b):
    B, N, T, D = embs.shape
    V = vis.shape[-1]
    K = W.shape[0]
    R = B * N
    L = 16

    info = plsc.get_sparse_core_info()
    NC, NS = info.num_cores, info.num_subcores
    NW = NC * NS
    RPW = R // NW

    embs3 = embs.reshape(R, T, D)
    masks2 = masks.reshape(R, T).astype(jnp.float32)
    vis_b = jnp.broadcast_to(
        jnp.transpose(vis.reshape(R, T, V), (0, 2, 1))[..., None],
        (R, V, T, L))
    mask_b = jnp.broadcast_to(masks2[:, None, :, None], (R, 1, T, L))
    vm = jnp.concatenate([vis_b, mask_b], axis=1).reshape(R, (V + 1) * T * L)

    mesh = plsc.VectorSubcoreMesh(core_axis_name="c", subcore_axis_name="s")
    sc_body = functools.partial(
        _sc_reduce_body, R=R, T=T, D=D, V=V, L=L, NC=NC, RPW=RPW,
        alpha=_ALPHA)
    feats = pl.kernel(
        sc_body,
        out_type=jax.ShapeDtypeStruct((R, D), jnp.float32),
        mesh=mesh,
        compiler_params=pltpu.CompilerParams(use_tc_tiling_on_sc=True),
        scratch_types=[
            pltpu.VMEM((2, T, D), jnp.float32),
            pltpu.VMEM((2 * (V + 1) * T * L,), jnp.float32),
            pltpu.VMEM((V * T * L,), jnp.float32),
            pltpu.VMEM((2 * T * L,), jnp.float32),
            pltpu.VMEM((D,), jnp.float32),
            pltpu.SemaphoreType.DMA,
            pltpu.SemaphoreType.DMA,
        ],
    )(embs3, vm)

    GL = 256
    out = pl.pallas_call(
        _tc_linear_body,
        grid=(R // GL,),
        in_specs=[
            pl.BlockSpec((GL, T), lambda i: (i, 0)),
            pl.BlockSpec((GL, D), lambda i: (i, 0)),
            pl.BlockSpec((K, D), lambda i: (0, 0)),
            pl.BlockSpec((1, K), lambda i: (0, 0)),
        ],
        out_specs=pl.BlockSpec((GL, K), lambda i: (i, 0)),
        out_shape=jax.ShapeDtypeStruct((R, K), jnp.float32),
    )(masks2, feats, W, b.reshape(1, K))
    return out.reshape(B, N, K)
